# baseline (speedup 1.0000x reference)
import functools

import jax
import jax.numpy as jnp
from jax import lax
from jax.experimental import pallas as pl
from jax.experimental.pallas import tpu as pltpu
from jax.experimental.pallas import tpu_sc as plsc

_B, _T, _D = 4096, 200, 64
_V = 1000000
_VFULL = (_V // 128) * 128
_NBLK = _VFULL // 128
_NSTAGE = _NBLK
_CH_A1 = 512
_NCH_A1 = _VFULL // _CH_A1
_BPC_A2 = 2
_NCH_A2 = _NBLK // _BPC_A2


@functools.cache
def _build_kernels():
    info = plsc.get_sparse_core_info()
    nc, ns = info.num_cores, info.num_subcores
    nw = nc * ns
    mesh = plsc.VectorSubcoreMesh(core_axis_name="c", subcore_axis_name="s")

    @functools.partial(
        pl.kernel,
        mesh=mesh,
        out_type=jax.ShapeDtypeStruct((_NSTAGE * _D, 128), jnp.float32),
        scratch_types=[
            pltpu.VMEM((_D, _CH_A1), jnp.float32),
            pltpu.VMEM((_D, _CH_A1), jnp.float32),
            pltpu.SemaphoreType.DMA,
            pltpu.SemaphoreType.DMA,
        ],
    )
    def detile(embt_hbm, stage_hbm, buf0, buf1, s0, s1):
        wid = lax.axis_index("s") * nc + lax.axis_index("c")

        def sfire(c, buf, sem):
            for k in range(_CH_A1 // 128):
                pltpu.async_copy(
                    buf.at[:, pl.ds(k * 128, 128)],
                    stage_hbm.at[pl.ds((c * (_CH_A1 // 128) + k) * _D, _D)],
                    sem)

        def swait(c, buf, sem):
            for k in range(_CH_A1 // 128):
                pltpu.make_async_copy(
                    buf.at[:, pl.ds(k * 128, 128)],
                    stage_hbm.at[pl.ds((c * (_CH_A1 // 128) + k) * _D, _D)],
                    sem).wait()

        def step(c, buf, sem, first):
            @pl.when(c < _NCH_A1)
            def _():
                if not first:
                    swait(c - 2 * nw, buf, sem)
                pltpu.sync_copy(
                    embt_hbm.at[:, pl.ds(c * _CH_A1, _CH_A1)], buf)
                sfire(c, buf, sem)

        step(wid, buf0, s0, True)
        step(wid + nw, buf1, s1, True)

        def body(j, carry):
            i0 = 2 * j + 2
            step(i0 * nw + wid, buf0, s0, False)
            step((i0 + 1) * nw + wid, buf1, s1, False)
            return carry

        nsuper = (-(-_NCH_A1 // nw) + 1) // 2
        lax.fori_loop(0, nsuper, body, 0)

        @pl.when(wid == 0)
        def _():
            swait(60 * nw, buf0, s0)
            swait(61 * nw, buf1, s1)

        @pl.when(wid > 0)
        def _():
            swait(59 * nw + wid, buf1, s1)
            swait(60 * nw + wid, buf0, s0)

    @functools.partial(
        pl.kernel,
        mesh=mesh,
        compiler_params=pltpu.CompilerParams(
            use_tc_tiling_on_sc=False, needs_layout_passes=False),
        out_type=jax.ShapeDtypeStruct((_V * _D // 128, 128), jnp.float32),
        scratch_types=[
            pltpu.VMEM((_BPC_A2 * _D, 128), jnp.float32),
            pltpu.VMEM((_BPC_A2 * _D, 128), jnp.float32),
            pltpu.VMEM((_BPC_A2 * _D, 128), jnp.float32),
            pltpu.VMEM((_BPC_A2 * _D, 128), jnp.float32),
            pltpu.SemaphoreType.DMA,
            pltpu.SemaphoreType.DMA,
            pltpu.SemaphoreType.DMA,
            pltpu.SemaphoreType.DMA,
        ],
    )
    def pack(stage_hbm, tail_hbm, out_hbm, in0, in1, pk0, pk1, gi0, gi1,
             so0, so1):
        wid = lax.axis_index("s") * nc + lax.axis_index("c")
        iota = lax.iota(jnp.int32, 16)
        rowv = [(iota + g * 16) >> 1 for g in range(8)]
        parcol = [((iota + g * 16) & 1) * 64 for g in range(8)]

        def gfire(c, buf, sem):
            pltpu.async_copy(
                stage_hbm.at[pl.ds(c * _BPC_A2 * _D, _BPC_A2 * _D)], buf, sem)

        def gwait(c, buf, sem):
            pltpu.make_async_copy(
                stage_hbm.at[pl.ds(c * _BPC_A2 * _D, _BPC_A2 * _D)], buf,
                sem).wait()

        def sfire(c, buf, sem):
            pltpu.async_copy(
                buf, out_hbm.at[pl.ds(c * _BPC_A2 * _D, _BPC_A2 * _D)], sem)

        def swait(c, buf, sem):
            pltpu.make_async_copy(
                buf, out_hbm.at[pl.ds(c * _BPC_A2 * _D, _BPC_A2 * _D)],
                sem).wait()

        rowvb = [[rowv[g] + blk * _D for g in range(8)]
                 for blk in range(_BPC_A2)]

        def transpose_chunk(src, dst):
            def dloop(d, carry):
                dsp = jnp.full((16,), d, jnp.int32)
                for blk in range(_BPC_A2):
                    for g in range(8):
                        v = src[blk * _D + d, pl.ds(g * 16, 16)]
                        plsc.store_scatter(
                            dst, [rowvb[blk][g], parcol[g] + dsp], v)
                return carry
            lax.fori_loop(0, _D, dloop, 0)

        def step(c, bin_, bpk, gsem, ssem, first):
            @pl.when(c < _NCH_A2)
            def _():
                gwait(c, bin_, gsem)
                if not first:
                    swait(c - 2 * nw, bpk, ssem)
                transpose_chunk(bin_, bpk)
                sfire(c, bpk, ssem)

                @pl.when(c + 2 * nw < _NCH_A2)
                def _():
                    gfire(c + 2 * nw, bin_, gsem)

        @pl.when(wid < _NCH_A2)
        def _():
            gfire(wid, in0, gi0)

        @pl.when(wid + nw < _NCH_A2)
        def _():
            gfire(wid + nw, in1, gi1)

        step(wid, in0, pk0, gi0, so0, True)
        step(wid + nw, in1, pk1, gi1, so1, True)

        def body(j, carry):
            i0 = 2 * j + 2
            step(i0 * nw + wid, in0, pk0, gi0, so0, False)
            step((i0 + 1) * nw + wid, in1, pk1, gi1, so1, False)
            return carry

        nsuper = (-(-_NCH_A2 // nw) + 1) // 2
        lax.fori_loop(0, nsuper, body, 0)

        @pl.when(wid < 2)
        def _():
            swait(121 * nw + wid, pk1, so1)
            swait(122 * nw + wid, pk0, so0)

        @pl.when(wid >= 2)
        def _():
            swait(120 * nw + wid, pk0, so0)
            swait(121 * nw + wid, pk1, so1)

        @pl.when(wid == nw - 1)
        def _():
            pltpu.sync_copy(tail_hbm, in0.at[pl.ds(0, 32)])
            pltpu.sync_copy(
                in0.at[pl.ds(0, 32)],
                out_hbm.at[pl.ds(_VFULL * _D // 128, 32)])

    @functools.partial(
        pl.kernel,
        mesh=mesh,
        compiler_params=pltpu.CompilerParams(
            use_tc_tiling_on_sc=False, needs_layout_passes=False),
        out_type=jax.ShapeDtypeStruct((_T, 8, 32, 8, 128), jnp.float32),
        scratch_types=[
            pltpu.VMEM((_T, 128), jnp.int32),
            pltpu.VMEM((_T, _D), jnp.float32),
            pltpu.VMEM((128, _D), jnp.float32),
            pltpu.VMEM((128, _D), jnp.float32),
            pltpu.VMEM((8, 8, 128), jnp.float32),
            pltpu.VMEM((8, 8, 128), jnp.float32),
            pltpu.SemaphoreType.DMA,
            pltpu.SemaphoreType.DMA,
            pltpu.SemaphoreType.DMA,
            pltpu.SemaphoreType.DMA,
        ],
    )
    def lookup(ct_hbm, emb_hbm, pos_hbm, out_hbm, idx_all, pos_v, rows0, rows1,
               tile0, tile1, g0, g1, s0, s1):
        wid = lax.axis_index("s") * nc + lax.axis_index("c")
        pltpu.sync_copy(ct_hbm.at[:, pl.ds(wid * 128, 128)], idx_all)
        pltpu.sync_copy(pos_hbm, pos_v)
        iota = lax.iota(jnp.int32, 16)
        dtc = [(iota + dj * 16) >> 3 for dj in range(4)]
        dsc = [(iota + dj * 16) & 7 for dj in range(4)]

        def gfire(t, rows, sem):
            pltpu.async_copy(emb_hbm.at[idx_all.at[t]], rows, sem)

        def gwait(t, rows, sem):
            pltpu.make_async_copy(emb_hbm.at[idx_all.at[t]], rows, sem).wait()

        def sfire(t, tile, sem):
            for dt in range(8):
                pltpu.async_copy(tile.at[dt], out_hbm.at[t, dt, wid], sem)

        def swait(t, tile, sem):
            for dt in range(8):
                pltpu.make_async_copy(
                    tile.at[dt], out_hbm.at[t, dt, wid], sem).wait()

        def add_transpose(t, rows, tile):
            pvec = [pos_v[t, pl.ds(dj * 16, 16)] for dj in range(4)]

            def bloop(bs, carry):
                bsp = jnp.full((16,), bs, jnp.int32)
                for dj in range(4):
                    v = rows[bs, pl.ds(dj * 16, 16)] + pvec[dj]
                    plsc.store_scatter(tile, [dtc[dj], dsc[dj], bsp], v)
                return carry

            lax.fori_loop(0, 128, bloop, 0)

        gfire(0, rows0, g0)

        def super_body(i, carry):
            t0 = 2 * i
            t1 = 2 * i + 1

            gfire(t1, rows1, g1)
            gwait(t0, rows0, g0)

            @pl.when(i >= 1)
            def _():
                swait(t0 - 2, tile0, s0)

            add_transpose(t0, rows0, tile0)
            sfire(t0, tile0, s0)

            @pl.when(i < _T // 2 - 1)
            def _():
                gfire(t0 + 2, rows0, g0)

            gwait(t1, rows1, g1)

            @pl.when(i >= 1)
            def _():
                swait(t0 - 1, tile1, s1)

            add_transpose(t1, rows1, tile1)
            sfire(t1, tile1, s1)
            return carry

        lax.fori_loop(0, _T // 2, super_body, 0)
        swait(_T - 2, tile0, s0)
        swait(_T - 1, tile1, s1)

    return detile, pack, lookup


def kernel(motion_codes, motion_pad_mask, emb_weight, pos_weight):
    detile, pack, lookup = _build_kernels()
    ct = motion_codes.T.astype(jnp.int32)
    stage = detile(emb_weight.T)
    tail = emb_weight[_VFULL:].reshape(32, 128)
    table = pack(stage, tail)
    out5 = lookup(ct, table.reshape(_V, _D), pos_weight)
    x = out5.transpose(2, 4, 0, 1, 3).reshape(_B, _T, _D)
    return x, motion_pad_mask


# speedup vs baseline: 1.5265x; 1.5265x over previous
import functools

import jax
import jax.numpy as jnp
from jax import lax
from jax.experimental import pallas as pl
from jax.experimental.pallas import tpu as pltpu
from jax.experimental.pallas import tpu_sc as plsc

_B, _T, _D = 4096, 200, 64
_V = 1000000
_VFULL = (_V // 128) * 128
_NBLK = _VFULL // 128
_NSTAGE = _NBLK + 1
_VPAD = _NSTAGE * 128
_CH_A1 = 512
_NCH_TC = -(-_V // _CH_A1)
_VPAD2 = _NCH_TC * _CH_A1
_CTOK = 2 * _T
_SUBCH = ((0, 128), (128, 128), (256, 72), (328, 72))


@functools.cache
def _build_kernels():
    info = plsc.get_sparse_core_info()
    nc, ns = info.num_cores, info.num_subcores
    nw = nc * ns
    mesh = plsc.VectorSubcoreMesh(core_axis_name="c", subcore_axis_name="s")

    @functools.partial(
        pl.pallas_call,
        grid=(_NCH_TC,),
        in_specs=[
            pl.BlockSpec((_CH_A1, _CH_A1), lambda j: (0, 0)),
            pl.BlockSpec((_D, _CH_A1), lambda j: (0, j)),
        ],
        out_specs=pl.BlockSpec((_CH_A1 // 2, 128), lambda j: (j, 0)),
        out_shape=jax.ShapeDtypeStruct((_VPAD2 // 2, 128), jnp.float32),
    )
    def repack(perm_ref, embt_ref, out_ref):
        z = lax.dot_general(
            perm_ref[...], embt_ref[...],
            dimension_numbers=(((1,), (1,)), ((), ())),
            preferred_element_type=jnp.float32)
        out_ref[:, 0:_D] = z[0:_CH_A1 // 2]
        out_ref[:, _D:128] = z[_CH_A1 // 2:_CH_A1]

    @functools.partial(
        pl.kernel,
        mesh=mesh,
        compiler_params=pltpu.CompilerParams(use_tc_tiling_on_sc=False),
        out_type=jax.ShapeDtypeStruct((_B * _T, 128), jnp.float32),
        scratch_types=[
            pltpu.VMEM((_B * _T // nw,), jnp.int32),
            pltpu.VMEM((_T, _D), jnp.float32),
            pltpu.VMEM((_CTOK, _D), jnp.float32),
            pltpu.VMEM((_CTOK, _D), jnp.float32),
            pltpu.SemaphoreType.DMA,
            pltpu.SemaphoreType.DMA,
            pltpu.SemaphoreType.DMA,
            pltpu.SemaphoreType.DMA,
        ],
    )
    def lookup(codes_hbm, emb_hbm, pos_hbm, out_hbm, idx_v, pos_v, buf0, buf1,
               g0, g1, s0, s1):
        tok_per_w = _B * _T // nw
        n_chunks = tok_per_w // _CTOK
        wid = lax.axis_index("s") * nc + lax.axis_index("c")
        tok_base = pl.multiple_of(wid * tok_per_w, 8)
        pltpu.sync_copy(codes_hbm.at[pl.ds(tok_base, tok_per_w)], idx_v)
        pltpu.sync_copy(pos_hbm, pos_v)

        def gfire(c, buf, sem):
            off = c * _CTOK
            for o, n in _SUBCH:
                pltpu.async_copy(
                    emb_hbm.at[idx_v.at[pl.ds(off + o, n)]],
                    buf.at[pl.ds(o, n)], sem)

        def gwait(c, buf, sem):
            off = c * _CTOK
            for o, n in _SUBCH:
                pltpu.make_async_copy(
                    emb_hbm.at[idx_v.at[pl.ds(off + o, n)]],
                    buf.at[pl.ds(o, n)], sem).wait()

        def sfire(c, buf, sem):
            pltpu.async_copy(
                buf,
                out_hbm.at[pl.ds(tok_base + c * _CTOK, _CTOK), pl.ds(0, _D)],
                sem)

        def swait(c, buf, sem):
            pltpu.make_async_copy(
                buf,
                out_hbm.at[pl.ds(tok_base + c * _CTOK, _CTOK), pl.ds(0, _D)],
                sem).wait()

        def add_chunk(buf):
            def tloop(t, carry):
                for dt in range(2):
                    tt = t * 2 + dt
                    for j in range(_D // 16):
                        sl = pl.ds(j * 16, 16)
                        pv = pos_v[tt, sl]
                        buf[tt, sl] = buf[tt, sl] + pv
                        buf[_T + tt, sl] = buf[_T + tt, sl] + pv
                return carry
            lax.fori_loop(0, _T // 2, tloop, 0)

        gfire(0, buf0, g0)

        def super_body(i, carry):
            c0 = 2 * i
            c1 = 2 * i + 1

            @pl.when(i >= 1)
            def _():
                swait(c0 - 1, buf1, s1)

            gfire(c1, buf1, g1)
            gwait(c0, buf0, g0)
            add_chunk(buf0)
            sfire(c0, buf0, s0)
            gwait(c1, buf1, g1)
            add_chunk(buf1)
            sfire(c1, buf1, s1)

            @pl.when(i < n_chunks // 2 - 1)
            def _():
                swait(c0, buf0, s0)
                gfire(c0 + 2, buf0, g0)

            return carry

        lax.fori_loop(0, n_chunks // 2, super_body, 0)
        swait(n_chunks - 2, buf0, s0)
        swait(n_chunks - 1, buf1, s1)

    return repack, lookup


import numpy as _np

_PERM = _np.zeros((_CH_A1, _CH_A1), _np.float32)
_PERM[_np.arange(_CH_A1 // 2), 2 * _np.arange(_CH_A1 // 2)] = 1.0
_PERM[_CH_A1 // 2 + _np.arange(_CH_A1 // 2),
      2 * _np.arange(_CH_A1 // 2) + 1] = 1.0


def kernel(motion_codes, motion_pad_mask, emb_weight, pos_weight):
    repack, lookup = _build_kernels()
    codes = motion_codes.reshape(-1).astype(jnp.int32)
    packed = repack(jnp.asarray(_PERM), emb_weight.T)
    table = packed.reshape(_VPAD2, _D)
    out2 = lookup(codes, table, pos_weight)
    x = out2.reshape(_B, _T, 128)[:, :, :_D]
    return x, motion_pad_mask


# speedup vs baseline: 2.6996x; 1.7684x over previous
import functools

import jax
import jax.numpy as jnp
from jax import lax
from jax.experimental import pallas as pl
from jax.experimental.pallas import tpu as pltpu
from jax.experimental.pallas import tpu_sc as plsc

_B, _T, _D = 4096, 200, 64
_V = 1000000
_VFULL = (_V // 128) * 128
_NBLK = _VFULL // 128
_NSTAGE = _NBLK + 1
_VPAD = _NSTAGE * 128
_CH_A1 = 512
_NCH_TC = -(-_V // _CH_A1)
_VPAD2 = _NCH_TC * _CH_A1
_CTOK = 2 * _T
_SUBCH = ((0, 128), (128, 128), (256, 72), (328, 72))


@functools.cache
def _build_kernels():
    info = plsc.get_sparse_core_info()
    nc, ns = info.num_cores, info.num_subcores
    nw = nc * ns
    mesh = plsc.VectorSubcoreMesh(core_axis_name="c", subcore_axis_name="s")

    @functools.partial(
        pl.pallas_call,
        grid=(_NCH_TC,),
        in_specs=[
            pl.BlockSpec((_CH_A1, _CH_A1), lambda j: (0, 0)),
            pl.BlockSpec((_D, _CH_A1), lambda j: (0, j)),
        ],
        out_specs=pl.BlockSpec((_CH_A1 // 2, 128), lambda j: (j, 0)),
        out_shape=jax.ShapeDtypeStruct((_VPAD2 // 2, 128), jnp.float32),
    )
    def repack(perm_ref, embt_ref, out_ref):
        z = lax.dot_general(
            perm_ref[...], embt_ref[...],
            dimension_numbers=(((1,), (1,)), ((), ())),
            preferred_element_type=jnp.float32)
        out_ref[:, 0:_D] = z[0:_CH_A1 // 2]
        out_ref[:, _D:128] = z[_CH_A1 // 2:_CH_A1]

    @functools.partial(
        pl.kernel,
        mesh=mesh,
        compiler_params=pltpu.CompilerParams(use_tc_tiling_on_sc=False),
        out_type=jax.ShapeDtypeStruct((_B * _T, 128), jnp.float32),
        scratch_types=[
            pltpu.VMEM((_B * _T // nw,), jnp.int32),
            pltpu.VMEM((_T, _D), jnp.float32),
            pltpu.VMEM((_CTOK, _D), jnp.float32),
            pltpu.VMEM((_CTOK, _D), jnp.float32),
            pltpu.SemaphoreType.DMA,
            pltpu.SemaphoreType.DMA,
            pltpu.SemaphoreType.DMA,
            pltpu.SemaphoreType.DMA,
        ],
    )
    def lookup(codes_hbm, emb_hbm, pos_hbm, out_hbm, idx_v, pos_v, buf0, buf1,
               g0, g1, s0, s1):
        tok_per_w = _B * _T // nw
        n_chunks = tok_per_w // _CTOK
        wid = lax.axis_index("s") * nc + lax.axis_index("c")
        tok_base = pl.multiple_of(wid * tok_per_w, 8)
        pltpu.sync_copy(codes_hbm.at[pl.ds(tok_base, tok_per_w)], idx_v)
        pltpu.sync_copy(pos_hbm, pos_v)

        def gfire(c, buf, sem):
            off = c * _CTOK
            for o, n in _SUBCH:
                pltpu.async_copy(
                    emb_hbm.at[idx_v.at[pl.ds(off + o, n)]],
                    buf.at[pl.ds(o, n)], sem)

        def gwait(c, buf, sem):
            off = c * _CTOK
            for o, n in _SUBCH:
                pltpu.make_async_copy(
                    emb_hbm.at[idx_v.at[pl.ds(off + o, n)]],
                    buf.at[pl.ds(o, n)], sem).wait()

        def sfire(c, buf, sem):
            pltpu.async_copy(
                buf,
                out_hbm.at[pl.ds(tok_base + c * _CTOK, _CTOK), pl.ds(0, _D)],
                sem)

        def swait(c, buf, sem):
            pltpu.make_async_copy(
                buf,
                out_hbm.at[pl.ds(tok_base + c * _CTOK, _CTOK), pl.ds(0, _D)],
                sem).wait()

        def add_chunk(buf):
            def tloop(t, carry):
                for dt in range(2):
                    tt = t * 2 + dt
                    for j in range(_D // 16):
                        sl = pl.ds(j * 16, 16)
                        pv = pos_v[tt, sl]
                        buf[tt, sl] = buf[tt, sl] + pv
                        buf[_T + tt, sl] = buf[_T + tt, sl] + pv
                return carry
            lax.fori_loop(0, _T // 2, tloop, 0)

        gfire(0, buf0, g0)

        def super_body(i, carry):
            c0 = 2 * i
            c1 = 2 * i + 1

            @pl.when(i >= 1)
            def _():
                swait(c0 - 1, buf1, s1)

            gfire(c1, buf1, g1)
            gwait(c0, buf0, g0)
            add_chunk(buf0)
            sfire(c0, buf0, s0)
            gwait(c1, buf1, g1)
            add_chunk(buf1)
            sfire(c1, buf1, s1)

            @pl.when(i < n_chunks // 2 - 1)
            def _():
                swait(c0, buf0, s0)
                gfire(c0 + 2, buf0, g0)

            return carry

        lax.fori_loop(0, n_chunks // 2, super_body, 0)
        swait(n_chunks - 2, buf0, s0)
        swait(n_chunks - 1, buf1, s1)

    return repack, lookup


import numpy as _np

_PERM = _np.zeros((_CH_A1, _CH_A1), _np.float32)
_PERM[_np.arange(_CH_A1 // 2), 2 * _np.arange(_CH_A1 // 2)] = 1.0
_PERM[_CH_A1 // 2 + _np.arange(_CH_A1 // 2),
      2 * _np.arange(_CH_A1 // 2) + 1] = 1.0


def kernel(motion_codes, motion_pad_mask, emb_weight, pos_weight):
    repack, lookup = _build_kernels()
    codes = motion_codes.reshape(-1).astype(jnp.int32)
    out2 = lookup(codes, emb_weight, pos_weight)
    x = out2.reshape(_B, _T, 128)[:, :, :_D]
    return x, motion_pad_mask


# speedup vs baseline: 2.7013x; 1.0006x over previous
import functools

import jax
import jax.numpy as jnp
from jax import lax
from jax.experimental import pallas as pl
from jax.experimental.pallas import tpu as pltpu
from jax.experimental.pallas import tpu_sc as plsc

_B, _T, _D = 4096, 200, 64
_CTOK = 2 * _T
_SUBCH = ((0, 128), (128, 128), (256, 72), (328, 72))


@functools.cache
def _build_kernels():
    info = plsc.get_sparse_core_info()
    nc, ns = info.num_cores, info.num_subcores
    nw = nc * ns
    mesh = plsc.VectorSubcoreMesh(core_axis_name="c", subcore_axis_name="s")

    @functools.partial(
        pl.kernel,
        mesh=mesh,
        compiler_params=pltpu.CompilerParams(use_tc_tiling_on_sc=False),
        out_type=jax.ShapeDtypeStruct((_B * _T, 128), jnp.float32),
        scratch_types=[
            pltpu.VMEM((_B * _T // nw,), jnp.int32),
            pltpu.VMEM((_T, _D), jnp.float32),
            pltpu.VMEM((_CTOK, _D), jnp.float32),
            pltpu.VMEM((_CTOK, _D), jnp.float32),
            pltpu.SemaphoreType.DMA,
            pltpu.SemaphoreType.DMA,
            pltpu.SemaphoreType.DMA,
            pltpu.SemaphoreType.DMA,
        ],
    )
    def lookup(codes_hbm, emb_hbm, pos_hbm, out_hbm, idx_v, pos_v, buf0, buf1,
               g0, g1, s0, s1):
        tok_per_w = _B * _T // nw
        n_chunks = tok_per_w // _CTOK
        wid = lax.axis_index("s") * nc + lax.axis_index("c")
        tok_base = pl.multiple_of(wid * tok_per_w, 8)
        pltpu.sync_copy(codes_hbm.at[pl.ds(tok_base, tok_per_w)], idx_v)
        pltpu.sync_copy(pos_hbm, pos_v)

        def gfire(c, buf, sem):
            off = c * _CTOK
            for o, n in _SUBCH:
                pltpu.async_copy(
                    emb_hbm.at[idx_v.at[pl.ds(off + o, n)]],
                    buf.at[pl.ds(o, n)], sem)

        def gwait(c, buf, sem):
            off = c * _CTOK
            for o, n in _SUBCH:
                pltpu.make_async_copy(
                    emb_hbm.at[idx_v.at[pl.ds(off + o, n)]],
                    buf.at[pl.ds(o, n)], sem).wait()

        def sfire(c, buf, sem):
            pltpu.async_copy(
                buf,
                out_hbm.at[pl.ds(tok_base + c * _CTOK, _CTOK), pl.ds(0, _D)],
                sem)

        def swait(c, buf, sem):
            pltpu.make_async_copy(
                buf,
                out_hbm.at[pl.ds(tok_base + c * _CTOK, _CTOK), pl.ds(0, _D)],
                sem).wait()

        def add_chunk(buf):
            def tloop(t, carry):
                for dt in range(2):
                    tt = t * 2 + dt
                    for j in range(_D // 16):
                        sl = pl.ds(j * 16, 16)
                        pv = pos_v[tt, sl]
                        buf[tt, sl] = buf[tt, sl] + pv
                        buf[_T + tt, sl] = buf[_T + tt, sl] + pv
                return carry
            lax.fori_loop(0, _T // 2, tloop, 0)

        gfire(0, buf0, g0)

        def super_body(i, carry):
            c0 = 2 * i
            c1 = 2 * i + 1

            @pl.when(i >= 1)
            def _():
                swait(c0 - 1, buf1, s1)

            gfire(c1, buf1, g1)
            gwait(c0, buf0, g0)
            add_chunk(buf0)
            sfire(c0, buf0, s0)
            gwait(c1, buf1, g1)
            add_chunk(buf1)
            sfire(c1, buf1, s1)

            @pl.when(i < n_chunks // 2 - 1)
            def _():
                swait(c0, buf0, s0)
                gfire(c0 + 2, buf0, g0)

            return carry

        lax.fori_loop(0, n_chunks // 2, super_body, 0)
        swait(n_chunks - 2, buf0, s0)
        swait(n_chunks - 1, buf1, s1)

    return lookup


def kernel(motion_codes, motion_pad_mask, emb_weight, pos_weight):
    lookup = _build_kernels()
    codes = motion_codes.reshape(-1).astype(jnp.int32)
    out2 = lookup(codes, emb_weight, pos_weight)
    x = out2.reshape(_B, _T, 128)[:, :, :_D]
    return x, motion_pad_mask
